# baseline (device time: 1151713 ns/iter reference)
import jax
import jax.numpy as jnp
from jax import lax
from jax.experimental import pallas as pl
from jax.experimental.pallas import tpu as pltpu

T = 4096
D = 2048
F = 4096
E_LOC = 4
C = 1280
S = E_LOC * C

BF = 512
NF = F // BF
BM1, BK1 = 320, 2048
NB1, NK1 = S // BM1, 2 * T // BK1
BM2, BK2 = 1024, 320
NB2, NK2 = 2 * T // BM2, S // BK2

_MESH = pl.DeviceIdType.MESH
_VMEM_LIM = pltpu.CompilerParams(vmem_limit_bytes=60 * 1024 * 1024)


def _xneighbor():
    return (1 - lax.axis_index("x"), lax.axis_index("y"))


def _partner_barrier(other):
    barrier = pltpu.get_barrier_semaphore()
    pl.semaphore_signal(barrier, inc=1, device_id=other, device_id_type=_MESH)
    pl.semaphore_wait(barrier, 1)


def _exchA_body(xb_ref, as_ref, xall_out, asall_out, sx, sa, rx, ra):
    other = _xneighbor()
    _partner_barrier(other)
    cx = pltpu.make_async_remote_copy(
        src_ref=xb_ref, dst_ref=xall_out.at[pl.ds(T, T), :],
        send_sem=sx, recv_sem=rx, device_id=other, device_id_type=_MESH)
    ca = pltpu.make_async_remote_copy(
        src_ref=as_ref, dst_ref=asall_out.at[pl.ds(32, 32), :],
        send_sem=sa, recv_sem=ra, device_id=other, device_id_type=_MESH)
    cx.start()
    ca.start()
    xall_out[:T, :] = xb_ref[...]
    asall_out[:32, :] = as_ref[...]
    cx.wait()
    ca.wait()


def _exchE_body(y_ref, y_out, s, r):
    other = _xneighbor()
    _partner_barrier(other)
    c = pltpu.make_async_remote_copy(
        src_ref=y_ref, dst_ref=y_out, send_sem=s, recv_sem=r,
        device_id=other, device_id_type=_MESH)
    c.start()
    c.wait()


def _p1_body(band_ref, idx_ref, x_ref, out_ref, acc):
    m, k = pl.program_id(0), pl.program_id(1)

    @pl.when(k == 0)
    def _():
        acc[...] = jnp.zeros_like(acc)

    lo, hi = band_ref[0, m], band_ref[1, m]

    @pl.when(jnp.logical_and(hi >= k * BK1, lo < (k + 1) * BK1))
    def _():
        iota = jax.lax.broadcasted_iota(jnp.int32, (BM1, BK1), 1) + k * BK1
        oh = (idx_ref[...] == iota).astype(jnp.bfloat16)
        acc[...] += jnp.dot(oh, x_ref[...], preferred_element_type=jnp.float32)

    @pl.when(k == NK1 - 1)
    def _():
        out_ref[...] = acc[...].astype(jnp.bfloat16)


def _p2_body(band_ref, idx_ref, y_ref, out_ref, acc):
    m, k = pl.program_id(0), pl.program_id(1)

    @pl.when(k == 0)
    def _():
        acc[...] = jnp.zeros_like(acc)

    lo, hi = band_ref[0, k], band_ref[1, k]

    @pl.when(jnp.logical_and(hi >= m * BM2, lo < (m + 1) * BM2))
    def _():
        iota = jax.lax.broadcasted_iota(jnp.int32, (BM2, BK2), 1) + k * BK2
        oh = (idx_ref[...] == iota).astype(jnp.bfloat16)
        acc[...] += jnp.dot(oh, y_ref[...], preferred_element_type=jnp.float32)

    @pl.when(k == NK2 - 1)
    def _():
        out_ref[...] = acc[...].astype(jnp.bfloat16)


def _moe_body(xb_ref, w1_ref, w2_ref, y_ref, acc):
    f = pl.program_id(1)

    @pl.when(f == 0)
    def _():
        acc[...] = jnp.zeros_like(acc)

    h = jnp.dot(xb_ref[...], w1_ref[...].astype(jnp.bfloat16),
                preferred_element_type=jnp.float32)
    h = jnp.maximum(h, 0.0).astype(jnp.bfloat16)
    acc[...] += jnp.dot(h, w2_ref[...].astype(jnp.bfloat16),
                        preferred_element_type=jnp.float32)

    @pl.when(f == NF - 1)
    def _():
        y_ref[...] = acc[...].astype(jnp.bfloat16)


def kernel(x, assign, W1, W2):
    my_x = lax.axis_index("x")

    xb = x.astype(jnp.bfloat16)
    assign2d = assign.reshape(32, 128)
    x_all, as_all = pl.pallas_call(
        _exchA_body,
        out_shape=[
            jax.ShapeDtypeStruct((2 * T, D), jnp.bfloat16),
            jax.ShapeDtypeStruct((64, 128), jnp.int32),
        ],
        in_specs=[pl.BlockSpec(memory_space=pltpu.VMEM)] * 2,
        out_specs=[pl.BlockSpec(memory_space=pltpu.VMEM)] * 2,
        scratch_shapes=[pltpu.SemaphoreType.DMA] * 4,
        compiler_params=pltpu.CompilerParams(
            collective_id=0, vmem_limit_bytes=60 * 1024 * 1024),
    )(xb, assign2d)
    assign_all = as_all.reshape(2 * T)

    local_e = assign_all - E_LOC * my_x
    key = jnp.where((local_e >= 0) & (local_e < E_LOC), local_e, E_LOC)
    sort_idx = jnp.argsort(key, stable=True).astype(jnp.int32)
    key_sorted = key[sort_idx]
    offsets = jnp.searchsorted(
        key_sorted, jnp.arange(E_LOC + 1, dtype=jnp.int32), side="left"
    ).astype(jnp.int32)
    s_ids = jnp.arange(S, dtype=jnp.int32)
    e_of_s = s_ids // C
    r_of_s = s_ids % C
    cnt_s = offsets[e_of_s + 1] - offsets[e_of_s]
    pos = jnp.clip(offsets[e_of_s] + r_of_s, 0, 2 * T - 1)
    slot_src = jnp.where(r_of_s < cnt_s, sort_idx[pos], 2 * T)
    slot_src2d = slot_src[:, None]
    rank = jnp.arange(2 * T, dtype=jnp.int32) - offsets[key_sorted]
    slot_sorted = jnp.where((key_sorted < E_LOC) & (rank < C),
                            key_sorted * C + rank, S).astype(jnp.int32)
    inv = jnp.argsort(sort_idx).astype(jnp.int32)
    slot_of_token = slot_sorted[inv]
    tok_slot2d = slot_of_token[:, None]
    def _bands(nblk, blk):
        v = slot_src.reshape(nblk, blk)
        ok = v < 2 * T
        lo = jnp.min(jnp.where(ok, v, 2 * T), axis=1)
        hi = jnp.max(jnp.where(ok, v, -1), axis=1)
        return jnp.stack([lo, hi]).astype(jnp.int32)
    band1 = _bands(NB1, BM1)
    band2 = _bands(NK2, BK2)

    Xbuf = pl.pallas_call(
        _p1_body,
        grid_spec=pltpu.PrefetchScalarGridSpec(
            num_scalar_prefetch=1,
            grid=(NB1, NK1),
            in_specs=[
                pl.BlockSpec((BM1, 1), lambda m, k, b: (m, 0)),
                pl.BlockSpec((BK1, D), lambda m, k, b: (k, 0)),
            ],
            out_specs=pl.BlockSpec((BM1, D), lambda m, k, b: (m, 0)),
            scratch_shapes=[pltpu.VMEM((BM1, D), jnp.float32)],
        ),
        out_shape=jax.ShapeDtypeStruct((S, D), jnp.bfloat16),
        compiler_params=_VMEM_LIM,
    )(band1, slot_src2d, x_all)

    Y = pl.pallas_call(
        _moe_body,
        grid=(E_LOC, NF),
        in_specs=[
            pl.BlockSpec((None, C, D), lambda e, f: (e, 0, 0)),
            pl.BlockSpec((None, D, BF), lambda e, f: (e, 0, f)),
            pl.BlockSpec((None, BF, D), lambda e, f: (e, f, 0)),
        ],
        out_specs=pl.BlockSpec((None, C, D), lambda e, f: (e, 0, 0)),
        out_shape=jax.ShapeDtypeStruct((E_LOC, C, D), jnp.bfloat16),
        scratch_shapes=[pltpu.VMEM((C, D), jnp.float32)],
        compiler_params=_VMEM_LIM,
    )(Xbuf.reshape(E_LOC, C, D), W1, W2)

    out_all = pl.pallas_call(
        _p2_body,
        grid_spec=pltpu.PrefetchScalarGridSpec(
            num_scalar_prefetch=1,
            grid=(NB2, NK2),
            in_specs=[
                pl.BlockSpec((BM2, 1), lambda m, k, b: (m, 0)),
                pl.BlockSpec((BK2, D), lambda m, k, b: (k, 0)),
            ],
            out_specs=pl.BlockSpec((BM2, D), lambda m, k, b: (m, 0)),
            scratch_shapes=[pltpu.VMEM((BM2, D), jnp.float32)],
        ),
        out_shape=jax.ShapeDtypeStruct((2 * T, D), jnp.bfloat16),
        compiler_params=_VMEM_LIM,
    )(band2, tok_slot2d, Y.reshape(S, D))

    recv = pl.pallas_call(
        _exchE_body,
        out_shape=jax.ShapeDtypeStruct((T, D), jnp.bfloat16),
        in_specs=[pl.BlockSpec(memory_space=pltpu.VMEM)],
        out_specs=pl.BlockSpec(memory_space=pltpu.VMEM),
        scratch_shapes=[pltpu.SemaphoreType.DMA] * 2,
        compiler_params=pltpu.CompilerParams(
            collective_id=1, vmem_limit_bytes=60 * 1024 * 1024),
    )(out_all[T:])

    return out_all[:T].astype(jnp.float32) + recv.astype(jnp.float32)


# device time: 938628 ns/iter; 1.2270x vs baseline; 1.2270x over previous
import jax
import jax.numpy as jnp
from jax import lax
from jax.experimental import pallas as pl
from jax.experimental.pallas import tpu as pltpu

T = 4096
D = 2048
F = 4096
E_LOC = 4
C = 1280
S = E_LOC * C

BF = 512
NF = F // BF
BM1, BK1 = 320, 2048
NB1, NK1 = S // BM1, 2 * T // BK1
BM2, BK2 = 1024, 320
NB2, NK2 = 2 * T // BM2, S // BK2

_MESH = pl.DeviceIdType.MESH
_VMEM_LIM = pltpu.CompilerParams(vmem_limit_bytes=60 * 1024 * 1024)


def _xneighbor():
    return (1 - lax.axis_index("x"), lax.axis_index("y"))


def _partner_barrier(other):
    barrier = pltpu.get_barrier_semaphore()
    pl.semaphore_signal(barrier, inc=1, device_id=other, device_id_type=_MESH)
    pl.semaphore_wait(barrier, 1)


def _exchA_body(xb_ref, as_ref, xall_out, asall_out, sx, sa, rx, ra):
    other = _xneighbor()
    _partner_barrier(other)
    cx = pltpu.make_async_remote_copy(
        src_ref=xb_ref, dst_ref=xall_out.at[pl.ds(T, T), :],
        send_sem=sx, recv_sem=rx, device_id=other, device_id_type=_MESH)
    ca = pltpu.make_async_remote_copy(
        src_ref=as_ref, dst_ref=asall_out.at[pl.ds(32, 32), :],
        send_sem=sa, recv_sem=ra, device_id=other, device_id_type=_MESH)
    cx.start()
    ca.start()
    xall_out[:T, :] = xb_ref[...]
    asall_out[:32, :] = as_ref[...]
    cx.wait()
    ca.wait()


def _exchE_body(y_ref, y_out, s, r):
    other = _xneighbor()
    _partner_barrier(other)
    c = pltpu.make_async_remote_copy(
        src_ref=y_ref, dst_ref=y_out, send_sem=s, recv_sem=r,
        device_id=other, device_id_type=_MESH)
    c.start()
    c.wait()


def _p1_body(band_ref, idx_ref, x_ref, out_ref):
    k = pl.program_id(0)

    @pl.when(k == 0)
    def _():
        out_ref[...] = jnp.zeros_like(out_ref)

    for m in range(NB1):
        lo, hi = band_ref[0, m], band_ref[1, m]

        @pl.when(jnp.logical_and(hi >= k * BK1, lo < (k + 1) * BK1))
        def _(m=m):
            idx = idx_ref[m * BM1:(m + 1) * BM1, :]
            iota = (jax.lax.broadcasted_iota(jnp.int32, (BM1, BK1), 1)
                    + k * BK1)
            oh = (idx == iota).astype(jnp.bfloat16)
            r = jnp.dot(oh, x_ref[...], preferred_element_type=jnp.float32)
            sl = pl.ds(m * BM1, BM1)
            out_ref[sl, :] += r.astype(jnp.bfloat16)


def _p2_body(band_ref, idx_ref, y_ref, mine_ref, theirs_ref):
    k = pl.program_id(0)

    @pl.when(k == 0)
    def _():
        mine_ref[...] = jnp.zeros_like(mine_ref)
        theirs_ref[...] = jnp.zeros_like(theirs_ref)

    lo, hi = band_ref[0, k], band_ref[1, k]
    for m in range(NB2):
        @pl.when(jnp.logical_and(hi >= m * BM2, lo < (m + 1) * BM2))
        def _(m=m):
            idx = idx_ref[m * BM2:(m + 1) * BM2, :]
            iota = (jax.lax.broadcasted_iota(jnp.int32, (BM2, BK2), 1)
                    + k * BK2)
            oh = (idx == iota).astype(jnp.bfloat16)
            r = jnp.dot(oh, y_ref[...], preferred_element_type=jnp.float32)
            half = NB2 // 2
            if m < half:
                sl = pl.ds(m * BM2, BM2)
                mine_ref[sl, :] += r.astype(jnp.bfloat16)
            else:
                sl = pl.ds((m - half) * BM2, BM2)
                theirs_ref[sl, :] += r.astype(jnp.bfloat16)


def _moe_body(xb_ref, w1_ref, w2_ref, y_ref, acc):
    f = pl.program_id(1)

    @pl.when(f == 0)
    def _():
        acc[...] = jnp.zeros_like(acc)

    h = jnp.dot(xb_ref[...], w1_ref[...].astype(jnp.bfloat16),
                preferred_element_type=jnp.float32)
    h = jnp.maximum(h, 0.0).astype(jnp.bfloat16)
    acc[...] += jnp.dot(h, w2_ref[...].astype(jnp.bfloat16),
                        preferred_element_type=jnp.float32)

    @pl.when(f == NF - 1)
    def _():
        y_ref[...] = acc[...].astype(jnp.bfloat16)


def _small_lut(keys, table):
    out = jnp.zeros(keys.shape, table.dtype)
    for e in range(table.shape[0]):
        out = jnp.where(keys == e, table[e], out)
    return out


def kernel(x, assign, W1, W2):
    my_x = lax.axis_index("x")

    xb = x.astype(jnp.bfloat16)
    assign2d = assign.reshape(32, 128)
    x_all, as_all = pl.pallas_call(
        _exchA_body,
        out_shape=[
            jax.ShapeDtypeStruct((2 * T, D), jnp.bfloat16),
            jax.ShapeDtypeStruct((64, 128), jnp.int32),
        ],
        in_specs=[pl.BlockSpec(memory_space=pltpu.VMEM)] * 2,
        out_specs=[pl.BlockSpec(memory_space=pltpu.VMEM)] * 2,
        scratch_shapes=[pltpu.SemaphoreType.DMA] * 4,
        compiler_params=pltpu.CompilerParams(
            collective_id=0, vmem_limit_bytes=60 * 1024 * 1024),
    )(xb, assign2d)
    assign_all = as_all.reshape(2 * T)

    local_e = assign_all - E_LOC * my_x
    key = jnp.where((local_e >= 0) & (local_e < E_LOC), local_e, E_LOC)
    sort_idx = jnp.argsort(key, stable=True).astype(jnp.int32)
    key_sorted = key[sort_idx]
    counts = jnp.sum(
        key[:, None] == jnp.arange(E_LOC + 1, dtype=jnp.int32)[None, :],
        axis=0, dtype=jnp.int32)
    offsets = jnp.concatenate(
        [jnp.zeros((1,), jnp.int32), jnp.cumsum(counts)]).astype(jnp.int32)
    s_ids = jnp.arange(S, dtype=jnp.int32)
    e_of_s = s_ids // C
    r_of_s = s_ids % C
    off_s = _small_lut(e_of_s, offsets[:E_LOC])
    cnt_s = _small_lut(e_of_s, counts[:E_LOC])
    pos = jnp.clip(off_s + r_of_s, 0, 2 * T - 1)
    slot_src = jnp.where(r_of_s < cnt_s, sort_idx[pos], 2 * T)
    slot_src2d = slot_src[:, None]
    rank = (jnp.arange(2 * T, dtype=jnp.int32)
            - _small_lut(key_sorted, offsets[:E_LOC + 1]))
    slot_sorted = jnp.where((key_sorted < E_LOC) & (rank < C),
                            key_sorted * C + rank, S).astype(jnp.int32)
    inv = jnp.argsort(sort_idx).astype(jnp.int32)
    slot_of_token = slot_sorted[inv]
    tok_slot2d = slot_of_token[:, None]
    def _bands(nblk, blk):
        v = slot_src.reshape(nblk, blk)
        ok = v < 2 * T
        lo = jnp.min(jnp.where(ok, v, 2 * T), axis=1)
        hi = jnp.max(jnp.where(ok, v, -1), axis=1)
        return jnp.stack([lo, hi]).astype(jnp.int32)
    band1 = _bands(NB1, BM1)
    band2 = _bands(NK2, BK2)

    Xbuf = pl.pallas_call(
        _p1_body,
        grid_spec=pltpu.PrefetchScalarGridSpec(
            num_scalar_prefetch=1,
            grid=(NK1,),
            in_specs=[
                pl.BlockSpec((S, 1), lambda k, b: (0, 0)),
                pl.BlockSpec((BK1, D), lambda k, b: (k, 0)),
            ],
            out_specs=pl.BlockSpec((S, D), lambda k, b: (0, 0)),
        ),
        out_shape=jax.ShapeDtypeStruct((S, D), jnp.bfloat16),
        compiler_params=_VMEM_LIM,
    )(band1, slot_src2d, x_all)

    Y = pl.pallas_call(
        _moe_body,
        grid=(E_LOC, NF),
        in_specs=[
            pl.BlockSpec((None, C, D), lambda e, f: (e, 0, 0)),
            pl.BlockSpec((None, D, BF), lambda e, f: (e, 0, f)),
            pl.BlockSpec((None, BF, D), lambda e, f: (e, f, 0)),
        ],
        out_specs=pl.BlockSpec((None, C, D), lambda e, f: (e, 0, 0)),
        out_shape=jax.ShapeDtypeStruct((E_LOC, C, D), jnp.bfloat16),
        scratch_shapes=[pltpu.VMEM((C, D), jnp.float32)],
        compiler_params=_VMEM_LIM,
    )(Xbuf.reshape(E_LOC, C, D), W1, W2)

    mine, theirs = pl.pallas_call(
        _p2_body,
        grid_spec=pltpu.PrefetchScalarGridSpec(
            num_scalar_prefetch=1,
            grid=(NK2,),
            in_specs=[
                pl.BlockSpec((2 * T, 1), lambda k, b: (0, 0)),
                pl.BlockSpec((BK2, D), lambda k, b: (k, 0)),
            ],
            out_specs=[
                pl.BlockSpec((T, D), lambda k, b: (0, 0)),
                pl.BlockSpec((T, D), lambda k, b: (0, 0)),
            ],
        ),
        out_shape=[
            jax.ShapeDtypeStruct((T, D), jnp.bfloat16),
            jax.ShapeDtypeStruct((T, D), jnp.bfloat16),
        ],
        compiler_params=_VMEM_LIM,
    )(band2, tok_slot2d, Y.reshape(S, D))

    recv = pl.pallas_call(
        _exchE_body,
        out_shape=jax.ShapeDtypeStruct((T, D), jnp.bfloat16),
        in_specs=[pl.BlockSpec(memory_space=pltpu.VMEM)],
        out_specs=pl.BlockSpec(memory_space=pltpu.VMEM),
        scratch_shapes=[pltpu.SemaphoreType.DMA] * 2,
        compiler_params=pltpu.CompilerParams(
            collective_id=1, vmem_limit_bytes=60 * 1024 * 1024),
    )(theirs)

    return mine.astype(jnp.float32) + recv.astype(jnp.float32)


# device time: 882704 ns/iter; 1.3048x vs baseline; 1.0634x over previous
import jax
import jax.numpy as jnp
from jax import lax
from jax.experimental import pallas as pl
from jax.experimental.pallas import tpu as pltpu

T = 4096
D = 2048
F = 4096
E_LOC = 4
C = 1280
S = E_LOC * C

BF = 512
NF = F // BF
BM1, BK1 = 320, 2048
NB1, NK1 = S // BM1, 2 * T // BK1
BM2, BK2 = 1024, 320
NB2, NK2 = 2 * T // BM2, S // BK2

_MESH = pl.DeviceIdType.MESH
_VMEM_LIM = pltpu.CompilerParams(vmem_limit_bytes=60 * 1024 * 1024)


def _xneighbor():
    return (1 - lax.axis_index("x"), lax.axis_index("y"))


def _partner_barrier(other):
    barrier = pltpu.get_barrier_semaphore()
    pl.semaphore_signal(barrier, inc=1, device_id=other, device_id_type=_MESH)
    pl.semaphore_wait(barrier, 1)


def _exchA_body(xb_ref, as_ref, xall_out, asall_out, sx, sa, rx, ra):
    other = _xneighbor()
    _partner_barrier(other)
    cx = pltpu.make_async_remote_copy(
        src_ref=xb_ref, dst_ref=xall_out.at[pl.ds(T, T), :],
        send_sem=sx, recv_sem=rx, device_id=other, device_id_type=_MESH)
    ca = pltpu.make_async_remote_copy(
        src_ref=as_ref, dst_ref=asall_out.at[pl.ds(32, 32), :],
        send_sem=sa, recv_sem=ra, device_id=other, device_id_type=_MESH)
    cx.start()
    ca.start()
    xall_out[:T, :] = xb_ref[...]
    asall_out[:32, :] = as_ref[...]
    cx.wait()
    ca.wait()


def _exchE_body(y_ref, y_out, s, r):
    other = _xneighbor()
    _partner_barrier(other)
    c = pltpu.make_async_remote_copy(
        src_ref=y_ref, dst_ref=y_out, send_sem=s, recv_sem=r,
        device_id=other, device_id_type=_MESH)
    c.start()
    c.wait()


def _p1_body(band_ref, idx_ref, x_ref, out_ref):
    k = pl.program_id(0)

    @pl.when(k == 0)
    def _():
        out_ref[...] = jnp.zeros_like(out_ref)

    row = idx_ref[...]
    for m in range(NB1):
        lo, hi = band_ref[0, m], band_ref[1, m]

        @pl.when(jnp.logical_and(hi >= k * BK1, lo < (k + 1) * BK1))
        def _(m=m):
            iota = (jax.lax.broadcasted_iota(jnp.int32, (BM1, BK1), 0)
                    + m * BM1)
            oh = (row == iota).astype(jnp.bfloat16)
            r = jnp.dot(oh, x_ref[...], preferred_element_type=jnp.float32)
            sl = pl.ds(m * BM1, BM1)
            out_ref[sl, :] += r.astype(jnp.bfloat16)


def _p2_body(band_ref, idx_ref, y_ref, mine_ref, theirs_ref):
    k = pl.program_id(0)

    @pl.when(k == 0)
    def _():
        mine_ref[...] = jnp.zeros_like(mine_ref)
        theirs_ref[...] = jnp.zeros_like(theirs_ref)

    lo, hi = band_ref[0, k], band_ref[1, k]
    for m in range(NB2):
        @pl.when(jnp.logical_and(hi >= m * BM2, lo < (m + 1) * BM2))
        def _(m=m):
            idx = idx_ref[m * BM2:(m + 1) * BM2, :]
            iota = (jax.lax.broadcasted_iota(jnp.int32, (BM2, BK2), 1)
                    + k * BK2)
            oh = (idx == iota).astype(jnp.bfloat16)
            r = jnp.dot(oh, y_ref[...], preferred_element_type=jnp.float32)
            half = NB2 // 2
            if m < half:
                sl = pl.ds(m * BM2, BM2)
                mine_ref[sl, :] += r.astype(jnp.bfloat16)
            else:
                sl = pl.ds((m - half) * BM2, BM2)
                theirs_ref[sl, :] += r.astype(jnp.bfloat16)


def _moe_body(xb_ref, w1_ref, w2_ref, y_ref, acc):
    f = pl.program_id(1)

    @pl.when(f == 0)
    def _():
        acc[...] = jnp.zeros_like(acc)

    h = jnp.dot(xb_ref[...], w1_ref[...].astype(jnp.bfloat16),
                preferred_element_type=jnp.float32)
    h = jnp.maximum(h, 0.0).astype(jnp.bfloat16)
    acc[...] += jnp.dot(h, w2_ref[...].astype(jnp.bfloat16),
                        preferred_element_type=jnp.float32)

    @pl.when(f == NF - 1)
    def _():
        y_ref[...] = acc[...].astype(jnp.bfloat16)


def kernel(x, assign, W1, W2):
    my_x = lax.axis_index("x")

    xb = x.astype(jnp.bfloat16)
    assign2d = assign.reshape(32, 128)
    x_all, as_all = pl.pallas_call(
        _exchA_body,
        out_shape=[
            jax.ShapeDtypeStruct((2 * T, D), jnp.bfloat16),
            jax.ShapeDtypeStruct((64, 128), jnp.int32),
        ],
        in_specs=[pl.BlockSpec(memory_space=pltpu.VMEM)] * 2,
        out_specs=[pl.BlockSpec(memory_space=pltpu.VMEM)] * 2,
        scratch_shapes=[pltpu.SemaphoreType.DMA] * 4,
        compiler_params=pltpu.CompilerParams(
            collective_id=0, vmem_limit_bytes=60 * 1024 * 1024),
    )(xb, assign2d)
    assign_all = as_all.reshape(2 * T)

    local_e = assign_all - E_LOC * my_x
    valid = (local_e >= 0) & (local_e < E_LOC)
    key = jnp.where(valid, local_e, E_LOC)
    ohk = (key[:, None] == jnp.arange(E_LOC, dtype=jnp.int32)[None, :]
           ).astype(jnp.int32)
    csum = jnp.cumsum(ohk, axis=0)
    rank = jnp.sum(ohk * (csum - 1), axis=1)
    slot_of_token = jnp.where(valid & (rank < C),
                              key * C + rank, S).astype(jnp.int32)
    tok_slot2d = slot_of_token[:, None]
    tok_slot_row = slot_of_token[None, :]
    t_iota = jnp.arange(2 * T, dtype=jnp.int32)[:, None]
    def _bands(nblk, blk):
        mask = (slot_of_token[:, None] // blk
                == jnp.arange(nblk, dtype=jnp.int32)[None, :])
        lo = jnp.min(jnp.where(mask, t_iota, 2 * T), axis=0)
        hi = jnp.max(jnp.where(mask, t_iota, -1), axis=0)
        return jnp.stack([lo, hi]).astype(jnp.int32)
    band1 = _bands(NB1, BM1)
    band2 = _bands(NK2, BK2)

    Xbuf = pl.pallas_call(
        _p1_body,
        grid_spec=pltpu.PrefetchScalarGridSpec(
            num_scalar_prefetch=1,
            grid=(NK1,),
            in_specs=[
                pl.BlockSpec((1, BK1), lambda k, b: (0, k)),
                pl.BlockSpec((BK1, D), lambda k, b: (k, 0)),
            ],
            out_specs=pl.BlockSpec((S, D), lambda k, b: (0, 0)),
        ),
        out_shape=jax.ShapeDtypeStruct((S, D), jnp.bfloat16),
        compiler_params=_VMEM_LIM,
    )(band1, tok_slot_row, x_all)

    Y = pl.pallas_call(
        _moe_body,
        grid=(E_LOC, NF),
        in_specs=[
            pl.BlockSpec((None, C, D), lambda e, f: (e, 0, 0)),
            pl.BlockSpec((None, D, BF), lambda e, f: (e, 0, f)),
            pl.BlockSpec((None, BF, D), lambda e, f: (e, f, 0)),
        ],
        out_specs=pl.BlockSpec((None, C, D), lambda e, f: (e, 0, 0)),
        out_shape=jax.ShapeDtypeStruct((E_LOC, C, D), jnp.bfloat16),
        scratch_shapes=[pltpu.VMEM((C, D), jnp.float32)],
        compiler_params=_VMEM_LIM,
    )(Xbuf.reshape(E_LOC, C, D), W1, W2)

    mine, theirs = pl.pallas_call(
        _p2_body,
        grid_spec=pltpu.PrefetchScalarGridSpec(
            num_scalar_prefetch=1,
            grid=(NK2,),
            in_specs=[
                pl.BlockSpec((2 * T, 1), lambda k, b: (0, 0)),
                pl.BlockSpec((BK2, D), lambda k, b: (k, 0)),
            ],
            out_specs=[
                pl.BlockSpec((T, D), lambda k, b: (0, 0)),
                pl.BlockSpec((T, D), lambda k, b: (0, 0)),
            ],
        ),
        out_shape=[
            jax.ShapeDtypeStruct((T, D), jnp.bfloat16),
            jax.ShapeDtypeStruct((T, D), jnp.bfloat16),
        ],
        compiler_params=_VMEM_LIM,
    )(band2, tok_slot2d, Y.reshape(S, D))

    recv = pl.pallas_call(
        _exchE_body,
        out_shape=jax.ShapeDtypeStruct((T, D), jnp.bfloat16),
        in_specs=[pl.BlockSpec(memory_space=pltpu.VMEM)],
        out_specs=pl.BlockSpec(memory_space=pltpu.VMEM),
        scratch_shapes=[pltpu.SemaphoreType.DMA] * 2,
        compiler_params=pltpu.CompilerParams(
            collective_id=1, vmem_limit_bytes=60 * 1024 * 1024),
    )(theirs)

    return mine.astype(jnp.float32) + recv.astype(jnp.float32)
